# Initial kernel scaffold; baseline (speedup 1.0000x reference)
import jax
import jax.numpy as jnp
from jax.experimental import pallas as pl


def kernel(image, codebook):
    raise NotImplementedError("write your pallas kernel here")


# speedup vs baseline: 1.2295x; 1.2295x over previous
import functools

import jax
import jax.numpy as jnp
from jax import lax
from jax.experimental import pallas as pl
from jax.experimental.pallas import tpu as pltpu
from jax.experimental.pallas import tpu_sc as plsc

_B = 8
_K = 8192
_C = 3
_H, _W = 512, 512
_L = (_H // _B) * (_W // _B)
_D = _B * _B * _C

_LT = 512
_KT = 2048


def _blockify(x, B):
    h, w, c = x.shape
    t = x.reshape(h // B, B, w // B, B, c)
    t = jnp.transpose(t, (0, 2, 4, 1, 3))
    return t.reshape(-1, B * B, c)


def _unblockify(blocks, image_shape, B):
    h, w, c = image_shape
    t = blocks.reshape(h // B, w // B, B, B, c)
    t = jnp.transpose(t, (0, 2, 1, 3, 4))
    return t.reshape(h, w, c)


def _argmin_body(bf_ref, cft_ref, out_ref):
    bf = bf_ref[:, :]
    bn = jnp.sum(bf * bf, axis=1, keepdims=True)

    def chunk(k, carry):
        best_val, best_idx = carry
        cfc = cft_ref[:, pl.ds(k * _KT, _KT)]
        ab = jax.lax.dot_general(
            bf, cfc, (((1,), (0,)), ((), ())),
            preferred_element_type=jnp.float32)
        cn = jnp.sum(cfc * cfc, axis=0, keepdims=True)
        dist = jnp.sqrt(jnp.maximum(bn + cn - 2.0 * ab, 0.0))
        m = jnp.min(dist, axis=1, keepdims=True)
        ii = lax.broadcasted_iota(jnp.int32, (_LT, _KT), 1) + k * _KT
        am = jnp.min(jnp.where(dist == m, ii, jnp.int32(2**30)),
                     axis=1, keepdims=True)
        upd = m < best_val
        return jnp.where(upd, m, best_val), jnp.where(upd, am, best_idx)

    init = (jnp.full((_LT, 1), jnp.inf, jnp.float32),
            jnp.zeros((_LT, 1), jnp.int32))
    _, best_idx = lax.fori_loop(0, _K // _KT, chunk, init)
    out_ref[:, :] = best_idx


_argmin_call = pl.pallas_call(
    _argmin_body,
    grid=(_L // _LT,),
    in_specs=[
        pl.BlockSpec((_LT, _D), lambda i: (i, 0)),
        pl.BlockSpec((_D, _K), lambda i: (0, 0)),
    ],
    out_specs=pl.BlockSpec((_LT, 1), lambda i: (i, 0)),
    out_shape=jax.ShapeDtypeStruct((_L, 1), jnp.int32),
)

_NC = 2
_NS = 16
_NW = _NC * _NS
_BPW = _L // _NW


_DP = 256


@functools.cache
def _make_sc_gather():
    @functools.partial(
        pl.kernel,
        mesh=plsc.VectorSubcoreMesh(core_axis_name="c", subcore_axis_name="s"),
        out_type=jax.ShapeDtypeStruct((_L, _DP), jnp.float32),
        scratch_types=[
            pltpu.VMEM((_BPW,), jnp.int32),
            pltpu.VMEM((_BPW, _DP), jnp.float32),
            pltpu.SemaphoreType.DMA,
        ],
    )
    def _sc_gather(table_hbm, idx_hbm, out_hbm, idx_v, rows_v, sem):
        wid = lax.axis_index("s") * _NC + lax.axis_index("c")
        base = wid * _BPW
        pltpu.sync_copy(idx_hbm.at[pl.ds(base, _BPW)], idx_v)
        pltpu.async_copy(table_hbm.at[idx_v], rows_v, sem).wait()
        pltpu.sync_copy(rows_v, out_hbm.at[pl.ds(base, _BPW)])

    return _sc_gather


def kernel(image, codebook):
    blocks2 = _blockify(_blockify(image, _B), _B)
    bf = blocks2.reshape(_L, _D)
    cf = codebook.reshape(_K, _D)
    closest = _argmin_call(bf, cf.T).reshape(_L)
    cf_pad = jnp.pad(cf, ((0, 0), (0, _DP - _D)))
    qrows = _make_sc_gather()(cf_pad, closest)[:, :_D]
    return _unblockify(qrows.reshape(_L, _B * _B, _C), image.shape, _B)
